# baseline (speedup 1.0000x reference)
import jax
import jax.numpy as jnp
import numpy as np
from jax.experimental import pallas as pl
from jax.experimental.shard_map import shard_map
from jax.sharding import Mesh, PartitionSpec as P

_N = 64
_MIN_DEPTH = 2.0
_MAX_DEPTH = 6.0
_BLKB = 4096
_LANES = 128


def _raysample_kernel(o_ref, d_ref, pts_ref, len_ref):
    step = (_MAX_DEPTH - _MIN_DEPTH) / (_N - 1)
    z = _MIN_DEPTH + step * jax.lax.broadcasted_iota(
        jnp.int32, (1, _N, 1), 1
    ).astype(jnp.float32)
    o = o_ref[...]
    d = d_ref[...]
    pts_ref[...] = o[:, None, :] + d[:, None, :] * z
    zl = _MIN_DEPTH + step * jax.lax.broadcasted_iota(
        jnp.int32, (_N, 1, 1), 0
    ).astype(jnp.float32)
    len_ref[...] = jnp.broadcast_to(zl, len_ref.shape)


def _raysample_block(o_t, d_t):
    Bl = o_t.shape[1]
    return pl.pallas_call(
        _raysample_kernel,
        grid=(Bl // _BLKB,),
        in_specs=[
            pl.BlockSpec((3, _BLKB), lambda i: (0, i)),
            pl.BlockSpec((3, _BLKB), lambda i: (0, i)),
        ],
        out_specs=[
            pl.BlockSpec((3, _N, _BLKB), lambda i: (0, 0, i)),
            pl.BlockSpec((_N, _BLKB // _LANES, _LANES), lambda i: (0, i, 0)),
        ],
        out_shape=[
            jax.ShapeDtypeStruct((3, _N, Bl), jnp.float32),
            jax.ShapeDtypeStruct((_N, Bl // _LANES, _LANES), jnp.float32),
        ],
    )(o_t, d_t)


@jax.jit
def kernel(origins, directions):
    B = origins.shape[0]
    o_t = origins.T
    d_t = directions.T
    devs = jax.devices()
    nd = 1
    while nd * 2 <= len(devs) and B % (nd * 2 * _BLKB) == 0:
        nd *= 2
    if nd > 1:
        mesh = Mesh(np.array(devs[:nd]), ("x",))
        pts_t, len_t = shard_map(
            _raysample_block,
            mesh=mesh,
            in_specs=(P(None, "x"), P(None, "x")),
            out_specs=(P(None, None, "x"), P(None, "x", None)),
            check_rep=False,
        )(o_t, d_t)
    else:
        pts_t, len_t = _raysample_block(o_t, d_t)
    pts = jnp.transpose(pts_t, (2, 1, 0))
    lengths = jax.lax.reshape(len_t, (B, _N, 1), dimensions=(1, 2, 0))
    return pts, lengths


# speedup vs baseline: 21.3502x; 21.3502x over previous
import jax
import jax.numpy as jnp
from jax.experimental import pallas as pl
from jax.experimental.pallas import tpu as pltpu

_N = 64
_MIN_DEPTH = 2.0
_MAX_DEPTH = 6.0
_BLKB = 8192
_LANES = 128


def _raysample_kernel(o_ref, d_ref, pts_ref, len_ref):
    step = (_MAX_DEPTH - _MIN_DEPTH) / (_N - 1)
    z = _MIN_DEPTH + step * jax.lax.broadcasted_iota(
        jnp.int32, (1, _N, 1), 1
    ).astype(jnp.float32)
    o = o_ref[...]
    d = d_ref[...]
    pts_ref[...] = o[:, None, :] + d[:, None, :] * z
    zl = _MIN_DEPTH + step * jax.lax.broadcasted_iota(
        jnp.int32, (_N, 1, 1), 0
    ).astype(jnp.float32)
    len_ref[...] = jnp.broadcast_to(zl, len_ref.shape)


@jax.jit
def kernel(origins, directions):
    B = origins.shape[0]
    o_t = origins.T
    d_t = directions.T
    pts_t, len_t = pl.pallas_call(
        _raysample_kernel,
        grid=(B // _BLKB,),
        in_specs=[
            pl.BlockSpec((3, _BLKB), lambda i: (0, i)),
            pl.BlockSpec((3, _BLKB), lambda i: (0, i)),
        ],
        out_specs=[
            pl.BlockSpec((3, _N, _BLKB), lambda i: (0, 0, i)),
            pl.BlockSpec((_N, _BLKB // _LANES, _LANES), lambda i: (0, i, 0)),
        ],
        out_shape=[
            jax.ShapeDtypeStruct((3, _N, B), jnp.float32),
            jax.ShapeDtypeStruct((_N, B // _LANES, _LANES), jnp.float32),
        ],
        compiler_params=pltpu.CompilerParams(
            dimension_semantics=("parallel",)
        ),
    )(o_t, d_t)
    pts = jnp.transpose(pts_t, (2, 1, 0))
    lengths = jax.lax.reshape(len_t, (B, _N, 1), dimensions=(1, 2, 0))
    return pts, lengths
